# baseline (speedup 1.0000x reference)
import functools

import jax
import jax.numpy as jnp
from jax.experimental import pallas as pl
from jax.experimental.pallas import tpu as pltpu

_PRE = 6000
_THR = 0.3
_K = 300
_NEG = float("-inf")
_LANES = 128
_SUB = 8


def _nms_body(scores_ref, coords_ref, out_ref, ws_ref, vols_ref):
    R = scores_ref.shape[0]
    NCH = R // _SUB
    scores = scores_ref[:]
    sbits = jax.lax.bitcast_convert_type(scores, jnp.int32)
    row = jax.lax.broadcasted_iota(jnp.int32, (R, _LANES), 0)
    col = jax.lax.broadcasted_iota(jnp.int32, (R, _LANES), 1)
    flat = row * _LANES + col
    big = jnp.int32(2 ** 30)

    def bs_val(_, lh):
        lo, hi = lh
        mid = lo + (hi - lo) // 2
        below = jnp.sum((sbits > mid).astype(jnp.int32)) < _PRE
        return (jnp.where(below, lo, mid + 1), jnp.where(below, mid, hi))

    tbits, _ = jax.lax.fori_loop(
        0, 31, bs_val, (jnp.int32(0), jnp.int32(0x3F800000)))
    gt = sbits > tbits
    eq = sbits == tbits
    need = _PRE - jnp.sum(gt.astype(jnp.int32))

    def bs_idx(_, lh):
        lo, hi = lh
        mid = lo + (hi - lo) // 2
        ok = jnp.sum((eq & (flat < mid)).astype(jnp.int32)) >= need
        return (jnp.where(ok, lo, mid + 1), jnp.where(ok, mid, hi))

    cut, _ = jax.lax.fori_loop(
        0, 16, bs_idx, (jnp.int32(0), jnp.int32(R * _LANES)))
    elig = gt | (eq & (flat < cut))

    ws_ref[:] = jnp.where(elig, scores, _NEG)
    vols_ref[:] = ((coords_ref[3] - coords_ref[0])
                   * (coords_ref[4] - coords_ref[1])
                   * (coords_ref[5] - coords_ref[2]))

    ii = (jax.lax.broadcasted_iota(jnp.int32, (_SUB, _LANES), 0) * _LANES
          + jax.lax.broadcasted_iota(jnp.int32, (_SUB, _LANES), 1))
    lane = jax.lax.broadcasted_iota(jnp.int32, (1, _LANES), 1)

    def comb_ordered(acc, cand):
        if acc is None:
            return cand
        c = acc[0] >= cand[0]
        return tuple(jnp.where(c, a, b) for a, b in zip(acc, cand))

    def comb_lex(a, b):
        c = (a[0] > b[0]) | ((a[0] == b[0]) & (a[1] < b[1]))
        return tuple(jnp.where(c, x, y) for x, y in zip(a, b))

    def load_chunk(c):
        sl = pl.ds(c * _SUB, _SUB)
        return tuple(coords_ref[k, sl, :] for k in range(6))

    accs = [None, None]
    for c in range(NCH):
        w = ws_ref[pl.ds(c * _SUB, _SUB), :]
        cand = (w, ii + c * _SUB * _LANES) + load_chunk(c)
        accs[c & 1] = comb_ordered(accs[c & 1], cand)
    carry0 = comb_lex(accs[0], accs[1])

    def winner(carry):
        v8, i8 = carry[0], carry[1]
        m = jnp.max(v8, keepdims=True)
        sel = jnp.min(jnp.where(v8 == m, i8, big), keepdims=True)
        wmask = (v8 == m) & (i8 == sel)
        coords = tuple(
            jnp.max(jnp.where(wmask, carry[2 + k], _NEG), keepdims=True)
            for k in range(6))
        return m, coords

    m0, coords0 = winner(carry0)

    def body(i, carry):
        m, coords = winner(carry)
        empty = m == _NEG
        sc = jnp.where(empty, m0, m)
        y1, x1, z1, y2, x2, z2 = (
            jnp.where(empty, c0, c) for c0, c in zip(coords0, coords))
        vol1 = (y2 - y1) * (x2 - x1) * (z2 - z1)

        accs = [None, None]
        for c in range(NCH):
            sl = pl.ds(c * _SUB, _SUB)
            w = ws_ref[sl, :]
            b0, b1, b2, b3, b4, b5 = load_chunk(c)
            vv = vols_ref[sl, :]
            inter = (jnp.maximum(jnp.minimum(y2, b3) - jnp.maximum(y1, b0), 0.0)
                     * jnp.maximum(jnp.minimum(x2, b4) - jnp.maximum(x1, b1), 0.0)
                     * jnp.maximum(jnp.minimum(z2, b5) - jnp.maximum(z1, b2), 0.0))
            iou = inter / ((vol1 + vv - inter) + 1e-8)
            neww = jnp.where(iou > _THR, _NEG, w)
            ws_ref[sl, :] = neww
            cand = (neww, ii + c * _SUB * _LANES, b0, b1, b2, b3, b4, b5)
            accs[c & 1] = comb_ordered(accs[c & 1], cand)

        v = jnp.full((1, _LANES), 0.0, jnp.float32)
        for j, val in enumerate((y1, x1, z1, y2, x2, z2, sc)):
            v = jnp.where(lane == j, val, v)
        out_ref[pl.ds(i, 1), :] = v
        return comb_lex(accs[0], accs[1])

    jax.lax.fori_loop(0, _K, body, carry0)


@jax.jit
def kernel(boxes, scores):
    n = scores.shape[0]
    r = (n + _LANES - 1) // _LANES
    r = (r + 7) // 8 * 8
    pad = r * _LANES - n
    scores_p = jnp.concatenate(
        [scores, jnp.full((pad,), _NEG, jnp.float32)]).reshape(r, _LANES)
    boxes_p = jnp.concatenate([boxes, jnp.zeros((pad, 6), jnp.float32)], axis=0)
    coords = boxes_p.T.reshape(6, r, _LANES)

    out = pl.pallas_call(
        _nms_body,
        out_shape=jax.ShapeDtypeStruct((304, _LANES), jnp.float32),
        scratch_shapes=[
            pltpu.VMEM((r, _LANES), jnp.float32),
            pltpu.VMEM((r, _LANES), jnp.float32),
        ],
    )(scores_p, coords)
    return out[:_K, :7]
